# Initial kernel scaffold; baseline (speedup 1.0000x reference)
import jax
import jax.numpy as jnp
from jax.experimental import pallas as pl


def kernel(user, media, user_table, media_table):
    raise NotImplementedError("write your pallas kernel here")


# speedup vs baseline: 5.8866x; 5.8866x over previous
import functools

import jax
import jax.numpy as jnp
from jax import lax
from jax.experimental import pallas as pl
from jax.experimental.pallas import tpu as pltpu, tpu_sc as plsc

B = 4096
LU = 20
LM = 50
E = 12
EP = 16
NC, NS = 2, 16
NW = NC * NS
BPW = B // NW
CB = 8
NCHUNK = BPW // CB
U_ROWS = CB * LU
M_ROWS = CB * LM
OUT_W = CB * LU * LM


def _body(user_r, media_r, ut_hbm, mt_hbm, out_hbm,
          idx_u, idx_m, ue_rows, me_rows, out_buf, sem):
    wid = lax.axis_index("s") * NC + lax.axis_index("c")
    iota = lax.iota(jnp.int32, 16)

    def chunk(c, carry):
        ur0 = wid * (BPW * LU // 80) + c * (U_ROWS // 80)
        mr0 = wid * (BPW * LM // 100) + c * (M_ROWS // 100)
        pltpu.sync_copy(user_r.at[pl.ds(ur0, 2)], idx_u)
        pltpu.sync_copy(media_r.at[pl.ds(mr0, 4)], idx_m)

        cps = []
        for r in range(2):
            cps.append(pltpu.async_copy(
                ut_hbm.at[idx_u.at[r]],
                ue_rows.at[pl.ds(r * 80, 80)], sem))
        for r in range(4):
            cps.append(pltpu.async_copy(
                mt_hbm.at[idx_m.at[r]],
                me_rows.at[pl.ds(r * 100, 100)], sem))
        for cp in cps:
            cp.wait()

        def batch(b, carry2):
            ub = b * LU
            mb = b * LM
            for ib in range(2):
                accs = [[jnp.zeros((16,), jnp.float32) for _ in range(4)]
                        for _ in range(10)]
                for e in range(12):
                    ecol = jnp.broadcast_to(jnp.int32(e), (16,))
                    mvec = [plsc.load_gather(
                                me_rows,
                                [iota + (mb + jc * 16), ecol])
                            for jc in range(4)]
                    for ii in range(10):
                        i = ib * 10 + ii
                        s = plsc.load_gather(
                            ue_rows,
                            [jnp.broadcast_to(ub + i, (16,)), ecol])
                        for jc in range(4):
                            accs[ii][jc] = accs[ii][jc] + s * mvec[jc]
                for ii in range(10):
                    i = ib * 10 + ii
                    for jc in range(4):
                        v = 1.0 / (1.0 + jnp.exp(-accs[ii][jc]))
                        out_buf[pl.ds(b * (LU * LM) + i * LM + jc * 16, 16)] = v
            return carry2

        lax.fori_loop(0, CB, batch, 0)

        base = (wid * NCHUNK + c) * OUT_W
        pltpu.sync_copy(out_buf.at[pl.ds(0, OUT_W)],
                        out_hbm.at[pl.ds(base, OUT_W)])
        return carry

    lax.fori_loop(0, NCHUNK, chunk, 0)


@jax.jit
def kernel(user, media, user_table, media_table):
    user_r = user.astype(jnp.int32).reshape(B * LU // 80, 80)
    media_r = media.astype(jnp.int32).reshape(B * LM // 100, 100)
    mesh = plsc.VectorSubcoreMesh(core_axis_name="c", subcore_axis_name="s",
                                  num_cores=NC, num_subcores=NS)
    out = pl.kernel(
        _body,
        out_type=jax.ShapeDtypeStruct((B * LU * LM,), jnp.float32),
        mesh=mesh,
        scratch_types=[
            pltpu.VMEM((2, 80), jnp.int32),
            pltpu.VMEM((4, 100), jnp.int32),
            pltpu.VMEM((U_ROWS, EP), jnp.float32),
            pltpu.VMEM((M_ROWS + 16, EP), jnp.float32),
            pltpu.VMEM((OUT_W + 16,), jnp.float32),
            pltpu.SemaphoreType.DMA,
        ],
        compiler_params=pltpu.CompilerParams(needs_layout_passes=False,
                                             use_tc_tiling_on_sc=False),
    )(user_r, media_r,
      jnp.pad(user_table, ((0, 0), (0, EP - E))),
      jnp.pad(media_table, ((0, 0), (0, EP - E))))
    return out.reshape(B, LU, LM)
